# Initial kernel scaffold; baseline (speedup 1.0000x reference)
import jax
import jax.numpy as jnp
from jax.experimental import pallas as pl


def kernel(x, global_x, params, edge_index):
    raise NotImplementedError("write your pallas kernel here")


# speedup vs baseline: 6.9682x; 6.9682x over previous
import functools

import jax
import jax.numpy as jnp
from jax import lax
from jax.experimental import pallas as pl
from jax.experimental.pallas import tpu as pltpu
from jax.experimental.pallas import tpu_sc as plsc

N = 10000
NP = 10240
E = 160000
EP = 163840
ROWS = EP // 128
NC, NS = 2, 16
NW = NC * NS
RW = ROWS // NW
SENT = N
F32 = jnp.float32


def _mesh():
    return plsc.VectorSubcoreMesh(
        core_axis_name="c", subcore_axis_name="s", num_cores=NC, num_subcores=NS
    )


_SC_PARAMS = pltpu.CompilerParams(use_tc_tiling_on_sc=False)


def _fill(buf, value, width):
    def row(i, carry):
        for j in range(width // 16):
            buf[i, pl.ds(j * 16, 16)] = jnp.full((16,), value, F32)
        return carry
    lax.fori_loop(0, 128, row, 0)


def _sc_conv(width):
    @functools.partial(
        pl.kernel,
        mesh=_mesh(),
        compiler_params=_SC_PARAMS,
        out_type=jax.ShapeDtypeStruct((NC, NP, width), F32),
        scratch_types=[
            pltpu.VMEM((RW, 128), jnp.int32),
            pltpu.VMEM((RW, 128), jnp.int32),
            pltpu.VMEM((128, width), F32),
            pltpu.VMEM((128, width), F32),
            pltpu.VMEM_SHARED((NP, width), F32),
            pltpu.SemaphoreType.DMA,
        ],
    )
    def k(table, src2d, dst2d, out, srcv, dstv, rows, zbuf, accum, sem):
        c = lax.axis_index("c")
        s = lax.axis_index("s")
        w = s * NC + c
        _fill(zbuf, 0.0, width)
        base = s * (NP // NS)
        for t in range((NP // NS) // 128):
            pltpu.sync_copy(zbuf, accum.at[pl.ds(base + t * 128, 128)])
        plsc.subcore_barrier()
        pltpu.sync_copy(src2d.at[pl.ds(w * RW, RW)], srcv)
        pltpu.sync_copy(dst2d.at[pl.ds(w * RW, RW)], dstv)
        for i in range(RW):
            pltpu.async_copy(table.at[srcv.at[i]], rows, sem).wait()
            pltpu.sync_copy(rows, accum.at[dstv.at[i]], add=True)
        plsc.subcore_barrier()
        pltpu.sync_copy(accum.at[pl.ds(base, NP // NS)],
                        out.at[c, pl.ds(base, NP // NS)])
    return k


def _sc_deg():
    width = 16
    @functools.partial(
        pl.kernel,
        mesh=_mesh(),
        compiler_params=_SC_PARAMS,
        out_type=jax.ShapeDtypeStruct((NC, NP, width), F32),
        scratch_types=[
            pltpu.VMEM((RW, 128), jnp.int32),
            pltpu.VMEM((128, width), F32),
            pltpu.VMEM((128, width), F32),
            pltpu.VMEM_SHARED((NP, width), F32),
        ],
    )
    def k(dst2d, out, dstv, rows, zbuf, accum):
        c = lax.axis_index("c")
        s = lax.axis_index("s")
        w = s * NC + c
        _fill(zbuf, 0.0, width)
        _fill(rows, 1.0, width)
        base = s * (NP // NS)
        for t in range((NP // NS) // 128):
            pltpu.sync_copy(zbuf, accum.at[pl.ds(base + t * 128, 128)])
        plsc.subcore_barrier()
        pltpu.sync_copy(dst2d.at[pl.ds(w * RW, RW)], dstv)
        for i in range(RW):
            pltpu.sync_copy(rows, accum.at[dstv.at[i]], add=True)
        plsc.subcore_barrier()
        pltpu.sync_copy(accum.at[pl.ds(base, NP // NS)],
                        out.at[c, pl.ds(base, NP // NS)])
    return k


def _sc_edge_gather():
    width = 64
    @functools.partial(
        pl.kernel,
        mesh=_mesh(),
        compiler_params=_SC_PARAMS,
        out_type=[jax.ShapeDtypeStruct((EP, width), F32),
                  jax.ShapeDtypeStruct((EP, width), F32)],
        scratch_types=[
            pltpu.VMEM((RW, 128), jnp.int32),
            pltpu.VMEM((RW, 128), jnp.int32),
            pltpu.VMEM((128, width), F32),
            pltpu.SemaphoreType.DMA,
        ],
    )
    def k(table, src2d, dst2d, out_s, out_d, srcv, dstv, rows, sem):
        c = lax.axis_index("c")
        s = lax.axis_index("s")
        w = s * NC + c
        pltpu.sync_copy(src2d.at[pl.ds(w * RW, RW)], srcv)
        pltpu.sync_copy(dst2d.at[pl.ds(w * RW, RW)], dstv)
        for i in range(RW):
            pltpu.async_copy(table.at[srcv.at[i]], rows, sem).wait()
            pltpu.sync_copy(rows, out_s.at[pl.ds((w * RW + i) * 128, 128)])
            pltpu.async_copy(table.at[dstv.at[i]], rows, sem).wait()
            pltpu.sync_copy(rows, out_d.at[pl.ds((w * RW + i) * 128, 128)])
    return k


def _k_dinv(degp):
    def body(p_ref, o_ref):
        deg = 1.0 + p_ref[0, 0:N, 0:1] + p_ref[1, 0:N, 0:1]
        o_ref[...] = lax.rsqrt(deg)
    return pl.pallas_call(
        body, out_shape=jax.ShapeDtypeStruct((N, 1), F32))(degp)


def _k_pre_init(x, w, dinv):
    def body(x_ref, w_ref, d_ref, o_ref):
        xw = jnp.dot(x_ref[...], w_ref[...], preferred_element_type=F32)
        o_ref[0:N, :] = xw * d_ref[...]
        o_ref[N:NP, :] = jnp.zeros((NP - N, w.shape[1]), F32)
    return pl.pallas_call(
        body, out_shape=jax.ShapeDtypeStruct((NP, w.shape[1]), F32))(x, w, dinv)


def _k_post_init(pp, xwp, b, dinv):
    def body(p_ref, xw_ref, b_ref, d_ref, o_ref):
        agg = p_ref[0, 0:N, :] + p_ref[1, 0:N, :] + xw_ref[0:N, :]
        o_ref[...] = agg * d_ref[...] + b_ref[...]
    return pl.pallas_call(
        body, out_shape=jax.ShapeDtypeStruct((N, 32), F32)
    )(pp, xwp, b.reshape(1, -1), dinv)


def _k_pre(T, G, B, Ws, dinv):
    H = T.shape[0]
    def body(t_ref, g_ref, b_ref, w_ref, d_ref, o_ref):
        cols = []
        d = d_ref[...]
        for j in range(H):
            h = t_ref[j]
            mu = jnp.mean(h, axis=0, keepdims=True)
            var = jnp.mean((h - mu) ** 2, axis=0, keepdims=True)
            z = ((h - mu) * lax.rsqrt(var + 1e-5) * g_ref[j][None, :]
                 + b_ref[j][None, :])
            z = jnp.maximum(z, 0.0)
            xw = jnp.dot(z, w_ref[j], preferred_element_type=F32)
            cols.append(xw * d)
        o_ref[0:N, :] = jnp.concatenate(cols, axis=1)
        o_ref[N:NP, :] = jnp.zeros((NP - N, 32 * H), F32)
    return pl.pallas_call(
        body, out_shape=jax.ShapeDtypeStruct((NP, 32 * H), F32)
    )(T, G, B, Ws, dinv)


def _k_post(pp, xwp, T, bc, dinv):
    H = T.shape[0]
    def body(p_ref, xw_ref, t_ref, b_ref, d_ref, o_ref):
        agg = p_ref[0, 0:N, :] + p_ref[1, 0:N, :] + xw_ref[0:N, :]
        agg = agg * d_ref[...]
        for j in range(H):
            o_ref[j] = (t_ref[j] + agg[:, 32 * j:32 * (j + 1)]
                        + b_ref[j][None, :])
    return pl.pallas_call(
        body, out_shape=jax.ShapeDtypeStruct((H, N, 32), F32)
    )(pp, xwp, T, bc, dinv)


def _k_pre_last(tp, tl, wp, wl, dinv):
    def body(tp_ref, tl_ref, wp_ref, wl_ref, d_ref, o_ref):
        a = jnp.dot(tp_ref[...], wp_ref[...], preferred_element_type=F32)
        b = jnp.dot(tl_ref[...], wl_ref[...], preferred_element_type=F32)
        cols = jnp.concatenate(
            [a * d_ref[...], b * d_ref[...], jnp.zeros((N, 14), F32)], axis=1)
        o_ref[0:N, :] = cols
        o_ref[N:NP, :] = jnp.zeros((NP - N, 16), F32)
    return pl.pallas_call(
        body, out_shape=jax.ShapeDtypeStruct((NP, 16), F32)
    )(tp, tl, wp, wl, dinv)


def _k_node_head(col, pp16, xwp16, dinv, p_last, final):
    def body(p_ref, xw_ref, d_ref, bl_ref, w1_ref, b1_ref, w2_ref, b2_ref, o_ref):
        t = (p_ref[0, 0:N, col:col + 1] + p_ref[1, 0:N, col:col + 1]
             + xw_ref[0:N, col:col + 1]) * d_ref[...] + bl_ref[0, 0]
        s = jnp.sum(t * w1_ref[...], axis=0, keepdims=True) + b1_ref[...]
        s = jnp.maximum(s, 0.0)
        logits = jnp.dot(s, w2_ref[...], preferred_element_type=F32) + b2_ref[...]
        m = jnp.max(logits, axis=1, keepdims=True)
        e = jnp.exp(logits - m)
        o_ref[...] = e / jnp.sum(e, axis=1, keepdims=True)
    return pl.pallas_call(
        body, out_shape=jax.ShapeDtypeStruct((1, N), F32)
    )(pp16, xwp16, dinv, p_last["b"].reshape(1, 1),
      final[0]["W"], final[0]["b"].reshape(1, -1),
      final[1]["W"], final[1]["b"].reshape(1, -1))


def _k_pack_edge(ta, tf):
    def body(a_ref, f_ref, o_ref):
        o_ref[0:N, :] = jnp.concatenate([a_ref[...], f_ref[...]], axis=1)
        o_ref[N:NP, :] = jnp.zeros((NP - N, 64), F32)
    return pl.pallas_call(
        body, out_shape=jax.ShapeDtypeStruct((NP, 64), F32))(ta, tf)


_EB = 2000
_EG = E // _EB


def _k_edge_l1(cbase, gs, gd, w1, b1):
    def body(gs_ref, gd_ref, w_ref, b_ref, z_ref, ps_ref, pq_ref):
        z = (jnp.dot(gs_ref[:, cbase:cbase + 32], w_ref[0:32, :],
                     preferred_element_type=F32)
             + jnp.dot(gd_ref[:, cbase:cbase + 32], w_ref[32:64, :],
                       preferred_element_type=F32) + b_ref[...])
        z = jnp.maximum(z, 0.0)
        z_ref[...] = z
        ps_ref[0] = jnp.sum(z, axis=0, keepdims=True)
        pq_ref[0] = jnp.sum(z * z, axis=0, keepdims=True)
    return pl.pallas_call(
        body,
        grid=(_EG,),
        in_specs=[
            pl.BlockSpec((_EB, 64), lambda i: (i, 0)),
            pl.BlockSpec((_EB, 64), lambda i: (i, 0)),
            pl.BlockSpec((64, 28), lambda i: (0, 0)),
            pl.BlockSpec((1, 28), lambda i: (0, 0)),
        ],
        out_specs=[
            pl.BlockSpec((_EB, 28), lambda i: (i, 0)),
            pl.BlockSpec((1, 1, 28), lambda i: (i, 0, 0)),
            pl.BlockSpec((1, 1, 28), lambda i: (i, 0, 0)),
        ],
        out_shape=[jax.ShapeDtypeStruct((E, 28), F32),
                   jax.ShapeDtypeStruct((_EG, 1, 28), F32),
                   jax.ShapeDtypeStruct((_EG, 1, 28), F32)],
    )(gs, gd, w1, b1.reshape(1, -1))


def _k_edge_l2(z1, ps, pq, g, bb, w2, b2, dout, sigmoid):
    def body(z_ref, ps_ref, pq_ref, g_ref, bb_ref, w_ref, b_ref, *o_refs):
        mu = jnp.sum(ps_ref[...], axis=0) / E
        ex2 = jnp.sum(pq_ref[...], axis=0) / E
        var = ex2 - mu * mu
        zn = (z_ref[...] - mu) * lax.rsqrt(var + 1e-5) * g_ref[...] + bb_ref[...]
        z = jnp.dot(zn, w_ref[...], preferred_element_type=F32) + b_ref[...]
        if sigmoid:
            o_refs[0][...] = 1.0 / (1.0 + jnp.exp(-z))
        else:
            z = jnp.maximum(z, 0.0)
            o_refs[0][...] = z
            o_refs[1][0] = jnp.sum(z, axis=0, keepdims=True)
            o_refs[2][0] = jnp.sum(z * z, axis=0, keepdims=True)
    din = z1.shape[1]
    out_specs = [pl.BlockSpec((_EB, dout), lambda i: (i, 0))]
    out_shape = [jax.ShapeDtypeStruct((E, dout), F32)]
    if not sigmoid:
        out_specs += [pl.BlockSpec((1, 1, dout), lambda i: (i, 0, 0))] * 2
        out_shape += [jax.ShapeDtypeStruct((_EG, 1, dout), F32)] * 2
    return pl.pallas_call(
        body,
        grid=(_EG,),
        in_specs=[
            pl.BlockSpec((_EB, din), lambda i: (i, 0)),
            pl.BlockSpec((_EG, 1, din), lambda i: (0, 0, 0)),
            pl.BlockSpec((_EG, 1, din), lambda i: (0, 0, 0)),
            pl.BlockSpec((1, din), lambda i: (0, 0)),
            pl.BlockSpec((1, din), lambda i: (0, 0)),
            pl.BlockSpec((din, dout), lambda i: (0, 0)),
            pl.BlockSpec((1, dout), lambda i: (0, 0)),
        ],
        out_specs=out_specs,
        out_shape=out_shape,
    )(z1, ps, pq, g.reshape(1, -1), bb.reshape(1, -1), w2, b2.reshape(1, -1))


def _k_edge_acc(z3, wf1):
    def body(z_ref, w_ref, o_ref):
        @pl.when(pl.program_id(0) == 0)
        def _():
            o_ref[...] = jnp.zeros_like(o_ref)
        o_ref[...] += jnp.sum(z_ref[...] * w_ref[...], axis=0, keepdims=True)
    return pl.pallas_call(
        body,
        grid=(_EG,),
        in_specs=[
            pl.BlockSpec((_EB, 1), lambda i: (i, 0)),
            pl.BlockSpec((_EB, 64), lambda i: (i, 0)),
        ],
        out_specs=pl.BlockSpec((1, 64), lambda i: (0, 0)),
        out_shape=jax.ShapeDtypeStruct((1, 64), F32),
    )(z3, wf1)


def _k_edge_logits(s_raw, bf1, wf2, bf2):
    L = wf2.shape[1]
    BL = 2048
    g = pl.cdiv(L, BL)
    def body(s_ref, b1_ref, w_ref, b2_ref, o_ref):
        s = jnp.maximum(s_ref[...] + b1_ref[...], 0.0)
        o_ref[...] = jnp.dot(s, w_ref[...], preferred_element_type=F32) + b2_ref[...]
    return pl.pallas_call(
        body,
        grid=(g,),
        in_specs=[
            pl.BlockSpec((1, 64), lambda i: (0, 0)),
            pl.BlockSpec((1, 64), lambda i: (0, 0)),
            pl.BlockSpec((64, BL), lambda i: (0, i)),
            pl.BlockSpec((1, BL), lambda i: (0, i)),
        ],
        out_specs=pl.BlockSpec((1, BL), lambda i: (0, i)),
        out_shape=jax.ShapeDtypeStruct((1, L), F32),
    )(s_raw, bf1.reshape(1, -1), wf2, bf2.reshape(1, -1))


def _k_softmax(logits):
    def body(x_ref, o_ref):
        x = x_ref[...]
        m = jnp.max(x, axis=1, keepdims=True)
        e = jnp.exp(x - m)
        o_ref[...] = e / jnp.sum(e, axis=1, keepdims=True)
    return pl.pallas_call(
        body, out_shape=jax.ShapeDtypeStruct(logits.shape, F32))(logits)


def _k_value(t, fc1, fc2):
    def body(t_ref, w1_ref, b1_ref, w2_ref, b2_ref, o_ref):
        v = jnp.mean(t_ref[...], axis=0, keepdims=True)
        z = jnp.maximum(
            jnp.dot(v, w1_ref[...], preferred_element_type=F32) + b1_ref[...], 0.0)
        u = jnp.dot(z, w2_ref[...], preferred_element_type=F32) + b2_ref[...]
        o_ref[...] = 1.0 / (1.0 + jnp.exp(-u))
    return pl.pallas_call(
        body, out_shape=jax.ShapeDtypeStruct((1, 6), F32)
    )(t, fc1["W"], fc1["b"].reshape(1, -1), fc2["W"], fc2["b"].reshape(1, -1))


_MAX_HEADS = 3


def _res_level(T, plist, dinv, src2d, dst2d):
    outs = []
    for lo in range(0, T.shape[0], _MAX_HEADS):
        Tg = T[lo:lo + _MAX_HEADS]
        pg = plist[lo:lo + _MAX_HEADS]
        H = Tg.shape[0]
        G = jnp.stack([p["bn"]["g"] for p in pg])
        B = jnp.stack([p["bn"]["b"] for p in pg])
        Ws = jnp.stack([p["conv"]["W"] for p in pg])
        bc = jnp.stack([p["conv"]["b"] for p in pg])
        xwp = _k_pre(Tg, G, B, Ws, dinv)
        pp = _sc_conv(32 * H)(xwp, src2d, dst2d)
        outs.append(_k_post(pp, xwp, Tg, bc, dinv))
    return jnp.concatenate(outs, axis=0) if len(outs) > 1 else outs[0]


def _edge_head(cbase, gs, gd, ep, final):
    z1, ps, pq = _k_edge_l1(cbase, gs, gd, ep["lin"][0]["W"], ep["lin"][0]["b"])
    z2, ps2, pq2 = _k_edge_l2(z1, ps, pq, ep["bn"][0]["g"], ep["bn"][0]["b"],
                              ep["lin"][1]["W"], ep["lin"][1]["b"], 28, False)
    z3 = _k_edge_l2(z2, ps2, pq2, ep["bn"][1]["g"], ep["bn"][1]["b"],
                    ep["lin"][2]["W"], ep["lin"][2]["b"], 1, True)[0]
    s_raw = _k_edge_acc(z3, final[0]["W"])
    logits = _k_edge_logits(s_raw, final[0]["b"], final[1]["W"], final[1]["b"])
    return _k_softmax(logits)


def kernel(x, global_x, params, edge_index):
    src0 = edge_index[0].astype(jnp.int32)
    dst0 = edge_index[1].astype(jnp.int32)
    pad = jnp.full((EP - E,), SENT, jnp.int32)
    src2d = jnp.concatenate([src0, pad]).reshape(ROWS, 128)
    dst2d = jnp.concatenate([dst0, pad]).reshape(ROWS, 128)

    degp = _sc_deg()(dst2d)
    dinv = _k_dinv(degp)

    xwp = _k_pre_init(x, params["conv_init"]["W"], dinv)
    pp = _sc_conv(32)(xwp, src2d, dst2d)
    h = _k_post_init(pp, xwp, params["conv_init"]["b"], dinv)

    T = h[None]
    for p in params["deep"]:
        T = _res_level(T, [p], dinv, src2d, dst2d)
    h = T[0]

    heads = [params["pick_res"], params["place_res"], params["attack_res"],
             params["fortify_res"], params["value_res"]]
    T5 = jnp.broadcast_to(h[None], (5, N, 32))
    for lvl in range(3):
        T5 = _res_level(T5, [hp[lvl] for hp in heads], dinv, src2d, dst2d)
    t_pick, t_place = T5[0], T5[1]
    T3 = T5[2:]
    T3 = _res_level(T3, [heads[2][3], heads[3][3], heads[4][3]],
                    dinv, src2d, dst2d)
    t_att, t_fort, t_val = T3[0], T3[1], T3[2]

    xwp16 = _k_pre_last(t_pick, t_place, params["pick_last"]["W"],
                        params["place_last"]["W"], dinv)
    pp16 = _sc_conv(16)(xwp16, src2d, dst2d)
    pick = _k_node_head(0, pp16, xwp16, dinv, params["pick_last"],
                        params["pick_final"])
    place = _k_node_head(1, pp16, xwp16, dinv, params["place_last"],
                         params["place_final"])

    etab = _k_pack_edge(t_att, t_fort)
    gs, gd = _sc_edge_gather()(etab, src2d, dst2d)
    attack = _edge_head(0, gs, gd, params["attack_edge"], params["attack_final"])
    fortify = _edge_head(32, gs, gd, params["fortify_edge"],
                         params["fortify_final"])

    v = _k_value(t_val, params["value_fc1"], params["value_fc2"]).reshape(6)
    return (pick, place, attack, fortify, v)


# speedup vs baseline: 7.6286x; 1.0948x over previous
import functools

import jax
import jax.numpy as jnp
from jax import lax
from jax.experimental import pallas as pl
from jax.experimental.pallas import tpu as pltpu
from jax.experimental.pallas import tpu_sc as plsc

N = 10000
NP = 10240
E = 160000
EP = 163840
ROWS = EP // 128
NC, NS = 2, 16
NW = NC * NS
RW = ROWS // NW
SENT = N
F32 = jnp.float32


def _mesh():
    return plsc.VectorSubcoreMesh(
        core_axis_name="c", subcore_axis_name="s", num_cores=NC, num_subcores=NS
    )


_SC_PARAMS = pltpu.CompilerParams(use_tc_tiling_on_sc=False)


def _fill(buf, value, width):
    def row(i, carry):
        for j in range(width // 16):
            buf[i, pl.ds(j * 16, 16)] = jnp.full((16,), value, F32)
        return carry
    lax.fori_loop(0, 128, row, 0)


_NBUF = 4
_LEAD = 2


def _sc_conv(width):
    @functools.partial(
        pl.kernel,
        mesh=_mesh(),
        compiler_params=_SC_PARAMS,
        out_type=jax.ShapeDtypeStruct((NC, NP, width), F32),
        scratch_types=[
            pltpu.VMEM((RW, 128), jnp.int32),
            pltpu.VMEM((RW, 128), jnp.int32),
            [pltpu.VMEM((128, width), F32) for _ in range(_NBUF)],
            pltpu.VMEM_SHARED((NP, width), F32),
            pltpu.SemaphoreType.DMA((_NBUF,)),
            pltpu.SemaphoreType.DMA((_NBUF,)),
        ],
    )
    def k(table, src2d, dst2d, out, srcv, dstv, rows, accum, gsem, ssem):
        c = lax.axis_index("c")
        s = lax.axis_index("s")
        w = s * NC + c
        _fill(rows[0], 0.0, width)
        base = s * (NP // NS)
        for t in range((NP // NS) // 128):
            pltpu.sync_copy(rows[0], accum.at[pl.ds(base + t * 128, 128)])
        plsc.subcore_barrier()
        pltpu.sync_copy(src2d.at[pl.ds(w * RW, RW)], srcv)
        pltpu.sync_copy(dst2d.at[pl.ds(w * RW, RW)], dstv)
        dg = [None] * RW
        dsc = [None] * RW
        for i in range(_LEAD):
            dg[i] = pltpu.async_copy(
                table.at[srcv.at[i]], rows[i % _NBUF], gsem.at[i % _NBUF])
        for i in range(RW):
            nxt = i + _LEAD
            if nxt < RW:
                if nxt - _NBUF >= 0:
                    dsc[nxt - _NBUF].wait()
                dg[nxt] = pltpu.async_copy(
                    table.at[srcv.at[nxt]], rows[nxt % _NBUF],
                    gsem.at[nxt % _NBUF])
            dg[i].wait()
            dsc[i] = pltpu.async_copy(
                rows[i % _NBUF], accum.at[dstv.at[i]], ssem.at[i % _NBUF],
                add=True)
        for i in range(max(0, RW - _NBUF), RW):
            dsc[i].wait()
        plsc.subcore_barrier()
        pltpu.sync_copy(accum.at[pl.ds(base, NP // NS)],
                        out.at[c, pl.ds(base, NP // NS)])
    return k


def _sc_deg():
    width = 16
    @functools.partial(
        pl.kernel,
        mesh=_mesh(),
        compiler_params=_SC_PARAMS,
        out_type=jax.ShapeDtypeStruct((NC, NP, width), F32),
        scratch_types=[
            pltpu.VMEM((RW, 128), jnp.int32),
            pltpu.VMEM((128, width), F32),
            pltpu.VMEM((128, width), F32),
            pltpu.VMEM_SHARED((NP, width), F32),
            pltpu.SemaphoreType.DMA,
        ],
    )
    def k(dst2d, out, dstv, rows, zbuf, accum, sem):
        c = lax.axis_index("c")
        s = lax.axis_index("s")
        w = s * NC + c
        _fill(zbuf, 0.0, width)
        _fill(rows, 1.0, width)
        base = s * (NP // NS)
        for t in range((NP // NS) // 128):
            pltpu.sync_copy(zbuf, accum.at[pl.ds(base + t * 128, 128)])
        plsc.subcore_barrier()
        pltpu.sync_copy(dst2d.at[pl.ds(w * RW, RW)], dstv)
        ds = [pltpu.async_copy(rows, accum.at[dstv.at[i]], sem, add=True)
              for i in range(RW)]
        for d in ds:
            d.wait()
        plsc.subcore_barrier()
        pltpu.sync_copy(accum.at[pl.ds(base, NP // NS)],
                        out.at[c, pl.ds(base, NP // NS)])
    return k


def _sc_edge_gather():
    width = 64
    @functools.partial(
        pl.kernel,
        mesh=_mesh(),
        compiler_params=_SC_PARAMS,
        out_type=[jax.ShapeDtypeStruct((EP, width), F32),
                  jax.ShapeDtypeStruct((EP, width), F32)],
        scratch_types=[
            pltpu.VMEM((RW, 128), jnp.int32),
            pltpu.VMEM((RW, 128), jnp.int32),
            [pltpu.VMEM((128, width), F32) for _ in range(_NBUF)],
            pltpu.SemaphoreType.DMA((_NBUF,)),
            pltpu.SemaphoreType.DMA((_NBUF,)),
        ],
    )
    def k(table, src2d, dst2d, out_s, out_d, srcv, dstv, rows, gsem, ssem):
        c = lax.axis_index("c")
        s = lax.axis_index("s")
        w = s * NC + c
        pltpu.sync_copy(src2d.at[pl.ds(w * RW, RW)], srcv)
        pltpu.sync_copy(dst2d.at[pl.ds(w * RW, RW)], dstv)
        NOP = 2 * RW
        def g_of(j):
            idx = srcv if j % 2 == 0 else dstv
            return table.at[idx.at[j // 2]]
        def o_of(j):
            dst = out_s if j % 2 == 0 else out_d
            return dst.at[pl.ds((w * RW + j // 2) * 128, 128)]
        dg = [None] * NOP
        dsc = [None] * NOP
        for j in range(_LEAD):
            dg[j] = pltpu.async_copy(g_of(j), rows[j % _NBUF],
                                     gsem.at[j % _NBUF])
        for j in range(NOP):
            nxt = j + _LEAD
            if nxt < NOP:
                if nxt - _NBUF >= 0:
                    dsc[nxt - _NBUF].wait()
                dg[nxt] = pltpu.async_copy(g_of(nxt), rows[nxt % _NBUF],
                                           gsem.at[nxt % _NBUF])
            dg[j].wait()
            dsc[j] = pltpu.async_copy(rows[j % _NBUF], o_of(j),
                                      ssem.at[j % _NBUF])
        for j in range(max(0, NOP - _NBUF), NOP):
            dsc[j].wait()
    return k


def _k_dinv(degp):
    def body(p_ref, o_ref):
        deg = 1.0 + p_ref[0, 0:N, 0:1] + p_ref[1, 0:N, 0:1]
        o_ref[...] = lax.rsqrt(deg)
    return pl.pallas_call(
        body, out_shape=jax.ShapeDtypeStruct((N, 1), F32))(degp)


def _k_pre_init(x, w, dinv):
    def body(x_ref, w_ref, d_ref, o_ref):
        xw = jnp.dot(x_ref[...], w_ref[...], preferred_element_type=F32)
        o_ref[0:N, :] = xw * d_ref[...]
        o_ref[N:NP, :] = jnp.zeros((NP - N, w.shape[1]), F32)
    return pl.pallas_call(
        body, out_shape=jax.ShapeDtypeStruct((NP, w.shape[1]), F32))(x, w, dinv)


def _k_post_init(pp, xwp, b, dinv):
    def body(p_ref, xw_ref, b_ref, d_ref, o_ref):
        agg = p_ref[0, 0:N, :] + p_ref[1, 0:N, :] + xw_ref[0:N, :]
        o_ref[...] = agg * d_ref[...] + b_ref[...]
    return pl.pallas_call(
        body, out_shape=jax.ShapeDtypeStruct((N, 32), F32)
    )(pp, xwp, b.reshape(1, -1), dinv)


def _k_pre(T, G, B, Ws, dinv):
    H = T.shape[0]
    def body(t_ref, g_ref, b_ref, w_ref, d_ref, o_ref):
        cols = []
        d = d_ref[...]
        for j in range(H):
            h = t_ref[j]
            mu = jnp.mean(h, axis=0, keepdims=True)
            var = jnp.mean((h - mu) ** 2, axis=0, keepdims=True)
            z = ((h - mu) * lax.rsqrt(var + 1e-5) * g_ref[j][None, :]
                 + b_ref[j][None, :])
            z = jnp.maximum(z, 0.0)
            xw = jnp.dot(z, w_ref[j], preferred_element_type=F32)
            cols.append(xw * d)
        o_ref[0:N, :] = jnp.concatenate(cols, axis=1)
        o_ref[N:NP, :] = jnp.zeros((NP - N, 32 * H), F32)
    return pl.pallas_call(
        body, out_shape=jax.ShapeDtypeStruct((NP, 32 * H), F32)
    )(T, G, B, Ws, dinv)


def _k_post(pp, xwp, T, bc, dinv):
    H = T.shape[0]
    def body(p_ref, xw_ref, t_ref, b_ref, d_ref, o_ref):
        agg = p_ref[0, 0:N, :] + p_ref[1, 0:N, :] + xw_ref[0:N, :]
        agg = agg * d_ref[...]
        for j in range(H):
            o_ref[j] = (t_ref[j] + agg[:, 32 * j:32 * (j + 1)]
                        + b_ref[j][None, :])
    return pl.pallas_call(
        body, out_shape=jax.ShapeDtypeStruct((H, N, 32), F32)
    )(pp, xwp, T, bc, dinv)


def _k_pre_last(tp, tl, wp, wl, dinv):
    def body(tp_ref, tl_ref, wp_ref, wl_ref, d_ref, o_ref):
        a = jnp.dot(tp_ref[...], wp_ref[...], preferred_element_type=F32)
        b = jnp.dot(tl_ref[...], wl_ref[...], preferred_element_type=F32)
        cols = jnp.concatenate(
            [a * d_ref[...], b * d_ref[...], jnp.zeros((N, 14), F32)], axis=1)
        o_ref[0:N, :] = cols
        o_ref[N:NP, :] = jnp.zeros((NP - N, 16), F32)
    return pl.pallas_call(
        body, out_shape=jax.ShapeDtypeStruct((NP, 16), F32)
    )(tp, tl, wp, wl, dinv)


def _k_node_head(col, pp16, xwp16, dinv, p_last, final):
    def body(p_ref, xw_ref, d_ref, bl_ref, w1_ref, b1_ref, w2_ref, b2_ref, o_ref):
        t = (p_ref[0, 0:N, col:col + 1] + p_ref[1, 0:N, col:col + 1]
             + xw_ref[0:N, col:col + 1]) * d_ref[...] + bl_ref[0, 0]
        s = jnp.sum(t * w1_ref[...], axis=0, keepdims=True) + b1_ref[...]
        s = jnp.maximum(s, 0.0)
        logits = jnp.dot(s, w2_ref[...], preferred_element_type=F32) + b2_ref[...]
        m = jnp.max(logits, axis=1, keepdims=True)
        e = jnp.exp(logits - m)
        o_ref[...] = e / jnp.sum(e, axis=1, keepdims=True)
    return pl.pallas_call(
        body, out_shape=jax.ShapeDtypeStruct((1, N), F32)
    )(pp16, xwp16, dinv, p_last["b"].reshape(1, 1),
      final[0]["W"], final[0]["b"].reshape(1, -1),
      final[1]["W"], final[1]["b"].reshape(1, -1))


def _k_pack_edge(ta, tf):
    def body(a_ref, f_ref, o_ref):
        o_ref[0:N, :] = jnp.concatenate([a_ref[...], f_ref[...]], axis=1)
        o_ref[N:NP, :] = jnp.zeros((NP - N, 64), F32)
    return pl.pallas_call(
        body, out_shape=jax.ShapeDtypeStruct((NP, 64), F32))(ta, tf)


_EB = 2000
_EG = E // _EB


def _k_edge_l1(cbase, gs, gd, w1, b1):
    def body(gs_ref, gd_ref, w_ref, b_ref, z_ref, ps_ref, pq_ref):
        z = (jnp.dot(gs_ref[:, cbase:cbase + 32], w_ref[0:32, :],
                     preferred_element_type=F32)
             + jnp.dot(gd_ref[:, cbase:cbase + 32], w_ref[32:64, :],
                       preferred_element_type=F32) + b_ref[...])
        z = jnp.maximum(z, 0.0)
        z_ref[...] = z
        ps_ref[0] = jnp.sum(z, axis=0, keepdims=True)
        pq_ref[0] = jnp.sum(z * z, axis=0, keepdims=True)
    return pl.pallas_call(
        body,
        grid=(_EG,),
        in_specs=[
            pl.BlockSpec((_EB, 64), lambda i: (i, 0)),
            pl.BlockSpec((_EB, 64), lambda i: (i, 0)),
            pl.BlockSpec((64, 28), lambda i: (0, 0)),
            pl.BlockSpec((1, 28), lambda i: (0, 0)),
        ],
        out_specs=[
            pl.BlockSpec((_EB, 28), lambda i: (i, 0)),
            pl.BlockSpec((1, 1, 28), lambda i: (i, 0, 0)),
            pl.BlockSpec((1, 1, 28), lambda i: (i, 0, 0)),
        ],
        out_shape=[jax.ShapeDtypeStruct((E, 28), F32),
                   jax.ShapeDtypeStruct((_EG, 1, 28), F32),
                   jax.ShapeDtypeStruct((_EG, 1, 28), F32)],
    )(gs, gd, w1, b1.reshape(1, -1))


def _k_edge_l2(z1, ps, pq, g, bb, w2, b2, dout, sigmoid):
    def body(z_ref, ps_ref, pq_ref, g_ref, bb_ref, w_ref, b_ref, *o_refs):
        mu = jnp.sum(ps_ref[...], axis=0) / E
        ex2 = jnp.sum(pq_ref[...], axis=0) / E
        var = ex2 - mu * mu
        zn = (z_ref[...] - mu) * lax.rsqrt(var + 1e-5) * g_ref[...] + bb_ref[...]
        z = jnp.dot(zn, w_ref[...], preferred_element_type=F32) + b_ref[...]
        if sigmoid:
            o_refs[0][...] = 1.0 / (1.0 + jnp.exp(-z))
        else:
            z = jnp.maximum(z, 0.0)
            o_refs[0][...] = z
            o_refs[1][0] = jnp.sum(z, axis=0, keepdims=True)
            o_refs[2][0] = jnp.sum(z * z, axis=0, keepdims=True)
    din = z1.shape[1]
    out_specs = [pl.BlockSpec((_EB, dout), lambda i: (i, 0))]
    out_shape = [jax.ShapeDtypeStruct((E, dout), F32)]
    if not sigmoid:
        out_specs += [pl.BlockSpec((1, 1, dout), lambda i: (i, 0, 0))] * 2
        out_shape += [jax.ShapeDtypeStruct((_EG, 1, dout), F32)] * 2
    return pl.pallas_call(
        body,
        grid=(_EG,),
        in_specs=[
            pl.BlockSpec((_EB, din), lambda i: (i, 0)),
            pl.BlockSpec((_EG, 1, din), lambda i: (0, 0, 0)),
            pl.BlockSpec((_EG, 1, din), lambda i: (0, 0, 0)),
            pl.BlockSpec((1, din), lambda i: (0, 0)),
            pl.BlockSpec((1, din), lambda i: (0, 0)),
            pl.BlockSpec((din, dout), lambda i: (0, 0)),
            pl.BlockSpec((1, dout), lambda i: (0, 0)),
        ],
        out_specs=out_specs,
        out_shape=out_shape,
    )(z1, ps, pq, g.reshape(1, -1), bb.reshape(1, -1), w2, b2.reshape(1, -1))


def _k_edge_acc(z3, wf1):
    def body(z_ref, w_ref, o_ref):
        @pl.when(pl.program_id(0) == 0)
        def _():
            o_ref[...] = jnp.zeros_like(o_ref)
        o_ref[...] += jnp.sum(z_ref[...] * w_ref[...], axis=0, keepdims=True)
    return pl.pallas_call(
        body,
        grid=(_EG,),
        in_specs=[
            pl.BlockSpec((_EB, 1), lambda i: (i, 0)),
            pl.BlockSpec((_EB, 64), lambda i: (i, 0)),
        ],
        out_specs=pl.BlockSpec((1, 64), lambda i: (0, 0)),
        out_shape=jax.ShapeDtypeStruct((1, 64), F32),
    )(z3, wf1)


def _k_edge_logits(s_raw, bf1, wf2, bf2):
    L = wf2.shape[1]
    BL = 2048
    g = pl.cdiv(L, BL)
    def body(s_ref, b1_ref, w_ref, b2_ref, o_ref):
        s = jnp.maximum(s_ref[...] + b1_ref[...], 0.0)
        o_ref[...] = jnp.dot(s, w_ref[...], preferred_element_type=F32) + b2_ref[...]
    return pl.pallas_call(
        body,
        grid=(g,),
        in_specs=[
            pl.BlockSpec((1, 64), lambda i: (0, 0)),
            pl.BlockSpec((1, 64), lambda i: (0, 0)),
            pl.BlockSpec((64, BL), lambda i: (0, i)),
            pl.BlockSpec((1, BL), lambda i: (0, i)),
        ],
        out_specs=pl.BlockSpec((1, BL), lambda i: (0, i)),
        out_shape=jax.ShapeDtypeStruct((1, L), F32),
    )(s_raw, bf1.reshape(1, -1), wf2, bf2.reshape(1, -1))


def _k_softmax(logits):
    def body(x_ref, o_ref):
        x = x_ref[...]
        m = jnp.max(x, axis=1, keepdims=True)
        e = jnp.exp(x - m)
        o_ref[...] = e / jnp.sum(e, axis=1, keepdims=True)
    return pl.pallas_call(
        body, out_shape=jax.ShapeDtypeStruct(logits.shape, F32))(logits)


def _k_value(t, fc1, fc2):
    def body(t_ref, w1_ref, b1_ref, w2_ref, b2_ref, o_ref):
        v = jnp.mean(t_ref[...], axis=0, keepdims=True)
        z = jnp.maximum(
            jnp.dot(v, w1_ref[...], preferred_element_type=F32) + b1_ref[...], 0.0)
        u = jnp.dot(z, w2_ref[...], preferred_element_type=F32) + b2_ref[...]
        o_ref[...] = 1.0 / (1.0 + jnp.exp(-u))
    return pl.pallas_call(
        body, out_shape=jax.ShapeDtypeStruct((1, 6), F32)
    )(t, fc1["W"], fc1["b"].reshape(1, -1), fc2["W"], fc2["b"].reshape(1, -1))


_MAX_HEADS = 3


def _res_level(T, plist, dinv, src2d, dst2d):
    outs = []
    for lo in range(0, T.shape[0], _MAX_HEADS):
        Tg = T[lo:lo + _MAX_HEADS]
        pg = plist[lo:lo + _MAX_HEADS]
        H = Tg.shape[0]
        G = jnp.stack([p["bn"]["g"] for p in pg])
        B = jnp.stack([p["bn"]["b"] for p in pg])
        Ws = jnp.stack([p["conv"]["W"] for p in pg])
        bc = jnp.stack([p["conv"]["b"] for p in pg])
        xwp = _k_pre(Tg, G, B, Ws, dinv)
        pp = _sc_conv(32 * H)(xwp, src2d, dst2d)
        outs.append(_k_post(pp, xwp, Tg, bc, dinv))
    return jnp.concatenate(outs, axis=0) if len(outs) > 1 else outs[0]


def _edge_head(cbase, gs, gd, ep, final):
    z1, ps, pq = _k_edge_l1(cbase, gs, gd, ep["lin"][0]["W"], ep["lin"][0]["b"])
    z2, ps2, pq2 = _k_edge_l2(z1, ps, pq, ep["bn"][0]["g"], ep["bn"][0]["b"],
                              ep["lin"][1]["W"], ep["lin"][1]["b"], 28, False)
    z3 = _k_edge_l2(z2, ps2, pq2, ep["bn"][1]["g"], ep["bn"][1]["b"],
                    ep["lin"][2]["W"], ep["lin"][2]["b"], 1, True)[0]
    s_raw = _k_edge_acc(z3, final[0]["W"])
    logits = _k_edge_logits(s_raw, final[0]["b"], final[1]["W"], final[1]["b"])
    return _k_softmax(logits)


def kernel(x, global_x, params, edge_index):
    src0 = edge_index[0].astype(jnp.int32)
    dst0 = edge_index[1].astype(jnp.int32)
    pad = jnp.full((EP - E,), SENT, jnp.int32)
    src2d = jnp.concatenate([src0, pad]).reshape(ROWS, 128)
    dst2d = jnp.concatenate([dst0, pad]).reshape(ROWS, 128)

    degp = _sc_deg()(dst2d)
    dinv = _k_dinv(degp)

    xwp = _k_pre_init(x, params["conv_init"]["W"], dinv)
    pp = _sc_conv(32)(xwp, src2d, dst2d)
    h = _k_post_init(pp, xwp, params["conv_init"]["b"], dinv)

    T = h[None]
    for p in params["deep"]:
        T = _res_level(T, [p], dinv, src2d, dst2d)
    h = T[0]

    heads = [params["pick_res"], params["place_res"], params["attack_res"],
             params["fortify_res"], params["value_res"]]
    T5 = jnp.broadcast_to(h[None], (5, N, 32))
    for lvl in range(3):
        T5 = _res_level(T5, [hp[lvl] for hp in heads], dinv, src2d, dst2d)
    t_pick, t_place = T5[0], T5[1]
    T3 = T5[2:]
    T3 = _res_level(T3, [heads[2][3], heads[3][3], heads[4][3]],
                    dinv, src2d, dst2d)
    t_att, t_fort, t_val = T3[0], T3[1], T3[2]

    xwp16 = _k_pre_last(t_pick, t_place, params["pick_last"]["W"],
                        params["place_last"]["W"], dinv)
    pp16 = _sc_conv(16)(xwp16, src2d, dst2d)
    pick = _k_node_head(0, pp16, xwp16, dinv, params["pick_last"],
                        params["pick_final"])
    place = _k_node_head(1, pp16, xwp16, dinv, params["place_last"],
                         params["place_final"])

    etab = _k_pack_edge(t_att, t_fort)
    gs, gd = _sc_edge_gather()(etab, src2d, dst2d)
    attack = _edge_head(0, gs, gd, params["attack_edge"], params["attack_final"])
    fortify = _edge_head(32, gs, gd, params["fortify_edge"],
                         params["fortify_final"])

    v = _k_value(t_val, params["value_fc1"], params["value_fc2"]).reshape(6)
    return (pick, place, attack, fortify, v)


# speedup vs baseline: 7.8429x; 1.0281x over previous
import functools

import jax
import jax.numpy as jnp
from jax import lax
from jax.experimental import pallas as pl
from jax.experimental.pallas import tpu as pltpu
from jax.experimental.pallas import tpu_sc as plsc

N = 10000
NP = 10240
E = 160000
EP = 163840
CH = 256
ROWS = EP // CH
NC, NS = 2, 16
NW = NC * NS
RW = ROWS // NW
SENT = N
F32 = jnp.float32


def _mesh():
    return plsc.VectorSubcoreMesh(
        core_axis_name="c", subcore_axis_name="s", num_cores=NC, num_subcores=NS
    )


_SC_PARAMS = pltpu.CompilerParams(use_tc_tiling_on_sc=False)


def _fill(buf, value, width, nrows):
    def row(i, carry):
        for j in range(width // 16):
            buf[i, pl.ds(j * 16, 16)] = jnp.full((16,), value, F32)
        return carry
    lax.fori_loop(0, nrows, row, 0)


def _nbuf_for(width):
    return 2 if width >= 96 else 4


def _sc_conv(width):
    nbuf = _nbuf_for(width)
    lead = min(2, nbuf - 1)
    @functools.partial(
        pl.kernel,
        mesh=_mesh(),
        compiler_params=_SC_PARAMS,
        out_type=jax.ShapeDtypeStruct((NC, NP, width), F32),
        scratch_types=[
            pltpu.VMEM((RW, CH), jnp.int32),
            pltpu.VMEM((RW, CH), jnp.int32),
            [pltpu.VMEM((CH, width), F32) for _ in range(nbuf)],
            pltpu.VMEM_SHARED((NP, width), F32),
            pltpu.SemaphoreType.DMA((nbuf,)),
            pltpu.SemaphoreType.DMA((nbuf,)),
        ],
    )
    def k(table, src2d, dst2d, out, srcv, dstv, rows, accum, gsem, ssem):
        c = lax.axis_index("c")
        s = lax.axis_index("s")
        w = s * NC + c
        _fill(rows[0], 0.0, width, CH)
        base = s * (NP // NS)
        for t in range((NP // NS) // 128):
            pltpu.sync_copy(rows[0].at[pl.ds(0, 128)],
                            accum.at[pl.ds(base + t * 128, 128)])
        plsc.subcore_barrier()
        pltpu.sync_copy(src2d.at[pl.ds(w * RW, RW)], srcv)
        pltpu.sync_copy(dst2d.at[pl.ds(w * RW, RW)], dstv)
        dg = [None] * RW
        dsc = [None] * RW
        for i in range(lead):
            dg[i] = pltpu.async_copy(
                table.at[srcv.at[i]], rows[i % nbuf], gsem.at[i % nbuf])
        for i in range(RW):
            nxt = i + lead
            if nxt < RW:
                if nxt - nbuf >= 0:
                    dsc[nxt - nbuf].wait()
                dg[nxt] = pltpu.async_copy(
                    table.at[srcv.at[nxt]], rows[nxt % nbuf],
                    gsem.at[nxt % nbuf])
            dg[i].wait()
            dsc[i] = pltpu.async_copy(
                rows[i % nbuf], accum.at[dstv.at[i]], ssem.at[i % nbuf],
                add=True)
        for i in range(max(0, RW - nbuf), RW):
            dsc[i].wait()
        plsc.subcore_barrier()
        pltpu.sync_copy(accum.at[pl.ds(base, NP // NS)],
                        out.at[c, pl.ds(base, NP // NS)])
    return k


def _sc_deg():
    width = 16
    @functools.partial(
        pl.kernel,
        mesh=_mesh(),
        compiler_params=_SC_PARAMS,
        out_type=jax.ShapeDtypeStruct((NC, NP, width), F32),
        scratch_types=[
            pltpu.VMEM((RW, CH), jnp.int32),
            pltpu.VMEM((CH, width), F32),
            pltpu.VMEM((CH, width), F32),
            pltpu.VMEM_SHARED((NP, width), F32),
            pltpu.SemaphoreType.DMA,
        ],
    )
    def k(dst2d, out, dstv, rows, zbuf, accum, sem):
        c = lax.axis_index("c")
        s = lax.axis_index("s")
        w = s * NC + c
        _fill(zbuf, 0.0, width, CH)
        _fill(rows, 1.0, width, CH)
        base = s * (NP // NS)
        for t in range((NP // NS) // 128):
            pltpu.sync_copy(zbuf.at[pl.ds(0, 128)],
                            accum.at[pl.ds(base + t * 128, 128)])
        plsc.subcore_barrier()
        pltpu.sync_copy(dst2d.at[pl.ds(w * RW, RW)], dstv)
        ds = [pltpu.async_copy(rows, accum.at[dstv.at[i]], sem, add=True)
              for i in range(RW)]
        for d in ds:
            d.wait()
        plsc.subcore_barrier()
        pltpu.sync_copy(accum.at[pl.ds(base, NP // NS)],
                        out.at[c, pl.ds(base, NP // NS)])
    return k


def _sc_edge_gather():
    width = 64
    @functools.partial(
        pl.kernel,
        mesh=_mesh(),
        compiler_params=_SC_PARAMS,
        out_type=[jax.ShapeDtypeStruct((EP, width), F32),
                  jax.ShapeDtypeStruct((EP, width), F32)],
        scratch_types=[
            pltpu.VMEM((RW, CH), jnp.int32),
            pltpu.VMEM((RW, CH), jnp.int32),
            [pltpu.VMEM((CH, width), F32) for _ in range(4)],
            pltpu.SemaphoreType.DMA((4,)),
            pltpu.SemaphoreType.DMA((4,)),
        ],
    )
    def k(table, src2d, dst2d, out_s, out_d, srcv, dstv, rows, gsem, ssem):
        c = lax.axis_index("c")
        s = lax.axis_index("s")
        w = s * NC + c
        pltpu.sync_copy(src2d.at[pl.ds(w * RW, RW)], srcv)
        pltpu.sync_copy(dst2d.at[pl.ds(w * RW, RW)], dstv)
        NOP = 2 * RW
        nbuf, lead = 4, 2
        def g_of(j):
            idx = srcv if j % 2 == 0 else dstv
            return table.at[idx.at[j // 2]]
        def o_of(j):
            dst = out_s if j % 2 == 0 else out_d
            return dst.at[pl.ds((w * RW + j // 2) * CH, CH)]
        dg = [None] * NOP
        dsc = [None] * NOP
        for j in range(lead):
            dg[j] = pltpu.async_copy(g_of(j), rows[j % nbuf],
                                     gsem.at[j % nbuf])
        for j in range(NOP):
            nxt = j + lead
            if nxt < NOP:
                if nxt - nbuf >= 0:
                    dsc[nxt - nbuf].wait()
                dg[nxt] = pltpu.async_copy(g_of(nxt), rows[nxt % nbuf],
                                           gsem.at[nxt % nbuf])
            dg[j].wait()
            dsc[j] = pltpu.async_copy(rows[j % nbuf], o_of(j),
                                      ssem.at[j % nbuf])
        for j in range(max(0, NOP - nbuf), NOP):
            dsc[j].wait()
    return k


def _k_dinv(degp):
    def body(p_ref, o_ref):
        deg = 1.0 + p_ref[0, 0:N, 0:1] + p_ref[1, 0:N, 0:1]
        o_ref[...] = lax.rsqrt(deg)
    return pl.pallas_call(
        body, out_shape=jax.ShapeDtypeStruct((N, 1), F32))(degp)


def _k_pre_init(x, w, dinv):
    def body(x_ref, w_ref, d_ref, o_ref):
        xw = jnp.dot(x_ref[...], w_ref[...], preferred_element_type=F32)
        o_ref[0:N, :] = xw * d_ref[...]
        o_ref[N:NP, :] = jnp.zeros((NP - N, w.shape[1]), F32)
    return pl.pallas_call(
        body, out_shape=jax.ShapeDtypeStruct((NP, w.shape[1]), F32))(x, w, dinv)


def _k_post_init(pp, xwp, b, dinv):
    def body(p_ref, xw_ref, b_ref, d_ref, o_ref):
        agg = p_ref[0, 0:N, :] + p_ref[1, 0:N, :] + xw_ref[0:N, :]
        o_ref[...] = agg * d_ref[...] + b_ref[...]
    return pl.pallas_call(
        body, out_shape=jax.ShapeDtypeStruct((N, 32), F32)
    )(pp, xwp, b.reshape(1, -1), dinv)


def _k_pre(T, G, B, Ws, dinv):
    H = T.shape[0]
    def body(t_ref, g_ref, b_ref, w_ref, d_ref, o_ref):
        cols = []
        d = d_ref[...]
        for j in range(H):
            h = t_ref[j]
            mu = jnp.mean(h, axis=0, keepdims=True)
            var = jnp.mean((h - mu) ** 2, axis=0, keepdims=True)
            z = ((h - mu) * lax.rsqrt(var + 1e-5) * g_ref[j][None, :]
                 + b_ref[j][None, :])
            z = jnp.maximum(z, 0.0)
            xw = jnp.dot(z, w_ref[j], preferred_element_type=F32)
            cols.append(xw * d)
        o_ref[0:N, :] = jnp.concatenate(cols, axis=1)
        o_ref[N:NP, :] = jnp.zeros((NP - N, 32 * H), F32)
    return pl.pallas_call(
        body, out_shape=jax.ShapeDtypeStruct((NP, 32 * H), F32)
    )(T, G, B, Ws, dinv)


def _k_post(pp, xwp, T, bc, dinv):
    H = T.shape[0]
    def body(p_ref, xw_ref, t_ref, b_ref, d_ref, o_ref):
        agg = p_ref[0, 0:N, :] + p_ref[1, 0:N, :] + xw_ref[0:N, :]
        agg = agg * d_ref[...]
        for j in range(H):
            o_ref[j] = (t_ref[j] + agg[:, 32 * j:32 * (j + 1)]
                        + b_ref[j][None, :])
    return pl.pallas_call(
        body, out_shape=jax.ShapeDtypeStruct((H, N, 32), F32)
    )(pp, xwp, T, bc, dinv)


def _k_pre_last(tp, tl, wp, wl, dinv):
    def body(tp_ref, tl_ref, wp_ref, wl_ref, d_ref, o_ref):
        a = jnp.dot(tp_ref[...], wp_ref[...], preferred_element_type=F32)
        b = jnp.dot(tl_ref[...], wl_ref[...], preferred_element_type=F32)
        cols = jnp.concatenate(
            [a * d_ref[...], b * d_ref[...], jnp.zeros((N, 14), F32)], axis=1)
        o_ref[0:N, :] = cols
        o_ref[N:NP, :] = jnp.zeros((NP - N, 16), F32)
    return pl.pallas_call(
        body, out_shape=jax.ShapeDtypeStruct((NP, 16), F32)
    )(tp, tl, wp, wl, dinv)


def _k_node_head(col, pp16, xwp16, dinv, p_last, final):
    def body(p_ref, xw_ref, d_ref, bl_ref, w1_ref, b1_ref, w2_ref, b2_ref, o_ref):
        t = (p_ref[0, 0:N, col:col + 1] + p_ref[1, 0:N, col:col + 1]
             + xw_ref[0:N, col:col + 1]) * d_ref[...] + bl_ref[0, 0]
        s = jnp.sum(t * w1_ref[...], axis=0, keepdims=True) + b1_ref[...]
        s = jnp.maximum(s, 0.0)
        logits = jnp.dot(s, w2_ref[...], preferred_element_type=F32) + b2_ref[...]
        m = jnp.max(logits, axis=1, keepdims=True)
        e = jnp.exp(logits - m)
        o_ref[...] = e / jnp.sum(e, axis=1, keepdims=True)
    return pl.pallas_call(
        body, out_shape=jax.ShapeDtypeStruct((1, N), F32)
    )(pp16, xwp16, dinv, p_last["b"].reshape(1, 1),
      final[0]["W"], final[0]["b"].reshape(1, -1),
      final[1]["W"], final[1]["b"].reshape(1, -1))


def _k_pack_edge(ta, tf):
    def body(a_ref, f_ref, o_ref):
        o_ref[0:N, :] = jnp.concatenate([a_ref[...], f_ref[...]], axis=1)
        o_ref[N:NP, :] = jnp.zeros((NP - N, 64), F32)
    return pl.pallas_call(
        body, out_shape=jax.ShapeDtypeStruct((NP, 64), F32))(ta, tf)


_EB = 2000
_EG = E // _EB


def _k_edge_l1(cbase, gs, gd, w1, b1):
    def body(gs_ref, gd_ref, w_ref, b_ref, z_ref, ps_ref, pq_ref):
        z = (jnp.dot(gs_ref[:, cbase:cbase + 32], w_ref[0:32, :],
                     preferred_element_type=F32)
             + jnp.dot(gd_ref[:, cbase:cbase + 32], w_ref[32:64, :],
                       preferred_element_type=F32) + b_ref[...])
        z = jnp.maximum(z, 0.0)
        z_ref[...] = z
        ps_ref[0] = jnp.sum(z, axis=0, keepdims=True)
        pq_ref[0] = jnp.sum(z * z, axis=0, keepdims=True)
    return pl.pallas_call(
        body,
        grid=(_EG,),
        in_specs=[
            pl.BlockSpec((_EB, 64), lambda i: (i, 0)),
            pl.BlockSpec((_EB, 64), lambda i: (i, 0)),
            pl.BlockSpec((64, 28), lambda i: (0, 0)),
            pl.BlockSpec((1, 28), lambda i: (0, 0)),
        ],
        out_specs=[
            pl.BlockSpec((_EB, 28), lambda i: (i, 0)),
            pl.BlockSpec((1, 1, 28), lambda i: (i, 0, 0)),
            pl.BlockSpec((1, 1, 28), lambda i: (i, 0, 0)),
        ],
        out_shape=[jax.ShapeDtypeStruct((E, 28), F32),
                   jax.ShapeDtypeStruct((_EG, 1, 28), F32),
                   jax.ShapeDtypeStruct((_EG, 1, 28), F32)],
    )(gs, gd, w1, b1.reshape(1, -1))


def _k_edge_l2(z1, ps, pq, g, bb, w2, b2, dout, sigmoid):
    def body(z_ref, ps_ref, pq_ref, g_ref, bb_ref, w_ref, b_ref, *o_refs):
        mu = jnp.sum(ps_ref[...], axis=0) / E
        ex2 = jnp.sum(pq_ref[...], axis=0) / E
        var = ex2 - mu * mu
        zn = (z_ref[...] - mu) * lax.rsqrt(var + 1e-5) * g_ref[...] + bb_ref[...]
        z = jnp.dot(zn, w_ref[...], preferred_element_type=F32) + b_ref[...]
        if sigmoid:
            o_refs[0][...] = 1.0 / (1.0 + jnp.exp(-z))
        else:
            z = jnp.maximum(z, 0.0)
            o_refs[0][...] = z
            o_refs[1][0] = jnp.sum(z, axis=0, keepdims=True)
            o_refs[2][0] = jnp.sum(z * z, axis=0, keepdims=True)
    din = z1.shape[1]
    out_specs = [pl.BlockSpec((_EB, dout), lambda i: (i, 0))]
    out_shape = [jax.ShapeDtypeStruct((E, dout), F32)]
    if not sigmoid:
        out_specs += [pl.BlockSpec((1, 1, dout), lambda i: (i, 0, 0))] * 2
        out_shape += [jax.ShapeDtypeStruct((_EG, 1, dout), F32)] * 2
    return pl.pallas_call(
        body,
        grid=(_EG,),
        in_specs=[
            pl.BlockSpec((_EB, din), lambda i: (i, 0)),
            pl.BlockSpec((_EG, 1, din), lambda i: (0, 0, 0)),
            pl.BlockSpec((_EG, 1, din), lambda i: (0, 0, 0)),
            pl.BlockSpec((1, din), lambda i: (0, 0)),
            pl.BlockSpec((1, din), lambda i: (0, 0)),
            pl.BlockSpec((din, dout), lambda i: (0, 0)),
            pl.BlockSpec((1, dout), lambda i: (0, 0)),
        ],
        out_specs=out_specs,
        out_shape=out_shape,
    )(z1, ps, pq, g.reshape(1, -1), bb.reshape(1, -1), w2, b2.reshape(1, -1))


def _k_edge_acc(z3, wf1):
    def body(z_ref, w_ref, o_ref):
        @pl.when(pl.program_id(0) == 0)
        def _():
            o_ref[...] = jnp.zeros_like(o_ref)
        o_ref[...] += jnp.sum(z_ref[...] * w_ref[...], axis=0, keepdims=True)
    return pl.pallas_call(
        body,
        grid=(_EG,),
        in_specs=[
            pl.BlockSpec((_EB, 1), lambda i: (i, 0)),
            pl.BlockSpec((_EB, 64), lambda i: (i, 0)),
        ],
        out_specs=pl.BlockSpec((1, 64), lambda i: (0, 0)),
        out_shape=jax.ShapeDtypeStruct((1, 64), F32),
    )(z3, wf1)


def _k_edge_logits(s_raw, bf1, wf2, bf2):
    L = wf2.shape[1]
    BL = 2048
    g = pl.cdiv(L, BL)
    def body(s_ref, b1_ref, w_ref, b2_ref, o_ref):
        s = jnp.maximum(s_ref[...] + b1_ref[...], 0.0)
        o_ref[...] = jnp.dot(s, w_ref[...], preferred_element_type=F32) + b2_ref[...]
    return pl.pallas_call(
        body,
        grid=(g,),
        in_specs=[
            pl.BlockSpec((1, 64), lambda i: (0, 0)),
            pl.BlockSpec((1, 64), lambda i: (0, 0)),
            pl.BlockSpec((64, BL), lambda i: (0, i)),
            pl.BlockSpec((1, BL), lambda i: (0, i)),
        ],
        out_specs=pl.BlockSpec((1, BL), lambda i: (0, i)),
        out_shape=jax.ShapeDtypeStruct((1, L), F32),
    )(s_raw, bf1.reshape(1, -1), wf2, bf2.reshape(1, -1))


def _k_softmax(logits):
    def body(x_ref, o_ref):
        x = x_ref[...]
        m = jnp.max(x, axis=1, keepdims=True)
        e = jnp.exp(x - m)
        o_ref[...] = e / jnp.sum(e, axis=1, keepdims=True)
    return pl.pallas_call(
        body, out_shape=jax.ShapeDtypeStruct(logits.shape, F32))(logits)


def _k_value(t, fc1, fc2):
    def body(t_ref, w1_ref, b1_ref, w2_ref, b2_ref, o_ref):
        v = jnp.mean(t_ref[...], axis=0, keepdims=True)
        z = jnp.maximum(
            jnp.dot(v, w1_ref[...], preferred_element_type=F32) + b1_ref[...], 0.0)
        u = jnp.dot(z, w2_ref[...], preferred_element_type=F32) + b2_ref[...]
        o_ref[...] = 1.0 / (1.0 + jnp.exp(-u))
    return pl.pallas_call(
        body, out_shape=jax.ShapeDtypeStruct((1, 6), F32)
    )(t, fc1["W"], fc1["b"].reshape(1, -1), fc2["W"], fc2["b"].reshape(1, -1))


_MAX_HEADS = 3


def _res_level(T, plist, dinv, src2d, dst2d):
    outs = []
    for lo in range(0, T.shape[0], _MAX_HEADS):
        Tg = T[lo:lo + _MAX_HEADS]
        pg = plist[lo:lo + _MAX_HEADS]
        H = Tg.shape[0]
        G = jnp.stack([p["bn"]["g"] for p in pg])
        B = jnp.stack([p["bn"]["b"] for p in pg])
        Ws = jnp.stack([p["conv"]["W"] for p in pg])
        bc = jnp.stack([p["conv"]["b"] for p in pg])
        xwp = _k_pre(Tg, G, B, Ws, dinv)
        pp = _sc_conv(32 * H)(xwp, src2d, dst2d)
        outs.append(_k_post(pp, xwp, Tg, bc, dinv))
    return jnp.concatenate(outs, axis=0) if len(outs) > 1 else outs[0]


def _edge_head(cbase, gs, gd, ep, final):
    z1, ps, pq = _k_edge_l1(cbase, gs, gd, ep["lin"][0]["W"], ep["lin"][0]["b"])
    z2, ps2, pq2 = _k_edge_l2(z1, ps, pq, ep["bn"][0]["g"], ep["bn"][0]["b"],
                              ep["lin"][1]["W"], ep["lin"][1]["b"], 28, False)
    z3 = _k_edge_l2(z2, ps2, pq2, ep["bn"][1]["g"], ep["bn"][1]["b"],
                    ep["lin"][2]["W"], ep["lin"][2]["b"], 1, True)[0]
    s_raw = _k_edge_acc(z3, final[0]["W"])
    logits = _k_edge_logits(s_raw, final[0]["b"], final[1]["W"], final[1]["b"])
    return _k_softmax(logits)


def kernel(x, global_x, params, edge_index):
    src0 = edge_index[0].astype(jnp.int32)
    dst0 = edge_index[1].astype(jnp.int32)
    pad = jnp.full((EP - E,), SENT, jnp.int32)
    src2d = jnp.concatenate([src0, pad]).reshape(ROWS, CH)
    dst2d = jnp.concatenate([dst0, pad]).reshape(ROWS, CH)

    degp = _sc_deg()(dst2d)
    dinv = _k_dinv(degp)

    xwp = _k_pre_init(x, params["conv_init"]["W"], dinv)
    pp = _sc_conv(32)(xwp, src2d, dst2d)
    h = _k_post_init(pp, xwp, params["conv_init"]["b"], dinv)

    T = h[None]
    for p in params["deep"]:
        T = _res_level(T, [p], dinv, src2d, dst2d)
    h = T[0]

    ha = [params["attack_res"], params["fortify_res"], params["value_res"]]
    hb = [params["pick_res"], params["place_res"]]
    TA = jnp.broadcast_to(h[None], (3, N, 32))
    TB = jnp.broadcast_to(h[None], (2, N, 32))
    for lvl in range(3):
        TA = _res_level(TA, [hp[lvl] for hp in ha], dinv, src2d, dst2d)
        TB = _res_level(TB, [hp[lvl] for hp in hb], dinv, src2d, dst2d)
    TA = _res_level(TA, [hp[3] for hp in ha], dinv, src2d, dst2d)
    t_att, t_fort, t_val = TA[0], TA[1], TA[2]
    t_pick, t_place = TB[0], TB[1]

    xwp16 = _k_pre_last(t_pick, t_place, params["pick_last"]["W"],
                        params["place_last"]["W"], dinv)
    pp16 = _sc_conv(16)(xwp16, src2d, dst2d)
    pick = _k_node_head(0, pp16, xwp16, dinv, params["pick_last"],
                        params["pick_final"])
    place = _k_node_head(1, pp16, xwp16, dinv, params["place_last"],
                         params["place_final"])

    etab = _k_pack_edge(t_att, t_fort)
    gs, gd = _sc_edge_gather()(etab, src2d, dst2d)
    attack = _edge_head(0, gs, gd, params["attack_edge"], params["attack_final"])
    fortify = _edge_head(32, gs, gd, params["fortify_edge"],
                         params["fortify_final"])

    v = _k_value(t_val, params["value_fc1"], params["value_fc2"]).reshape(6)
    return (pick, place, attack, fortify, v)
